# Initial kernel scaffold; baseline (speedup 1.0000x reference)
import functools

import jax
import jax.numpy as jnp
from jax import lax
from jax.experimental import pallas as pl
from jax.experimental.pallas import tpu as pltpu
from jax.experimental.pallas import tpu_sc as plsc

_L = 16
_CP = 96


def _sc_gather(table, bx, by, A, NJ, NB, hgrid, wgrid):
    NR = A * NJ
    n_chunks = NR // _L
    jch = NJ // _L
    mesh = plsc.VectorSubcoreMesh(core_axis_name="c", subcore_axis_name="s")
    n_sub = mesh.num_subcores

    @functools.partial(
        pl.kernel,
        out_type=jax.ShapeDtypeStruct((NR, _CP), jnp.float32),
        mesh=mesh,
        scratch_types=[
            pltpu.VMEM((_L,), jnp.float32),
            pltpu.VMEM((_L,), jnp.float32),
            pltpu.VMEM((_L,), jnp.int32),
            pltpu.VMEM((_L, _CP), jnp.float32),
            pltpu.SemaphoreType.DMA,
        ],
    )
    def gather_k(table_hbm, bx_hbm, by_hbm, out_hbm, bx_v, by_v, idx_v, rows_v, sem):
        wid = lax.axis_index("c") * n_sub + lax.axis_index("s")

        @pl.when(wid < n_chunks)
        def _():
            a = wid // jch
            jbase = pl.multiple_of((wid % jch) * _L, _L)
            pltpu.sync_copy(bx_hbm.at[pl.ds(jbase, _L)], bx_v)
            pltpu.sync_copy(by_hbm.at[pl.ds(jbase, _L)], by_v)
            bxv = bx_v[...]
            byv = by_v[...]
            yv = (byv * jnp.float32(hgrid)).astype(jnp.int32)
            xv = (bxv * jnp.float32(wgrid)).astype(jnp.int32)
            jv = jbase + lax.iota(jnp.int32, _L)
            bv = jv // NB
            idx_v[...] = ((bv * A + a) * hgrid + yv) * wgrid + xv
            pltpu.async_copy(table_hbm.at[idx_v], rows_v, sem).wait()
            obase = pl.multiple_of(wid * _L, _L)
            pltpu.sync_copy(rows_v, out_hbm.at[pl.ds(obase, _L)])

    return gather_k(table, bx, by)


def _loss_body(NR, NJ, NC, hgrid, wgrid,
               g_ref, p_ref, pt_ref, a_ref, l_ref, c_ref, o_ref):
    g = g_ref[...]
    p = p_ref[...]
    pt = pt_ref[...]
    anc = a_ref[...]
    lab = l_ref[...]
    lobj, lbox, lclass = c_ref[0], c_ref[1], c_ref[2]

    bx, by = p[:, 0:1], p[:, 1:2]
    bw, bh = p[:, 2:3], p[:, 3:4]
    hf = jnp.float32(hgrid)
    wf = jnp.float32(wgrid)
    yf = jnp.floor(by * hf)
    xf = jnp.floor(bx * wf)
    cellc = yf * wf + xf
    cellr = jnp.floor(pt[1:2, :] * hf) * wf + jnp.floor(pt[0:1, :] * wf)

    jr = lax.broadcasted_iota(jnp.int32, (NR, NR), 0) % NJ
    jc = lax.broadcasted_iota(jnp.int32, (NR, NR), 1) % NJ
    clash = jnp.logical_and(cellr == cellc, jc < jr)
    free = 1.0 - jnp.max(clash.astype(jnp.float32), axis=1, keepdims=True)

    xrel = (bx - xf / wf) * wf
    yrel = (by - yf / hf) * hf
    wc = bw / anc[:, 0:1]
    hc = bh / anc[:, 1:2]

    col = lax.broadcasted_iota(jnp.int32, (NR, _CP), 1)
    z0 = jnp.sum(jnp.where(col == 0, g, 0.0), axis=1, keepdims=True)
    t = -z0
    obj = jnp.maximum(t, 0.0) + jnp.log1p(jnp.exp(-jnp.abs(t)))
    tgt = (jnp.where(col == 1, xrel, 0.0) + jnp.where(col == 2, yrel, 0.0)
           + jnp.where(col == 3, wc, 0.0) + jnp.where(col == 4, hc, 0.0))
    boxmask = jnp.logical_and(col >= 1, col <= 4)
    mse = jnp.sum(jnp.where(boxmask, (g - tgt) ** 2, 0.0),
                  axis=1, keepdims=True) * 0.25
    cmask = jnp.logical_and(col >= 5, col < 5 + NC)
    m = jnp.max(jnp.where(cmask, g, jnp.float32(-1e30)), axis=1, keepdims=True)
    se = jnp.sum(jnp.where(cmask, jnp.exp(g - m), 0.0), axis=1, keepdims=True)
    lse = m + jnp.log(se)
    zlab = jnp.sum(jnp.where(col == lab + 5, g, 0.0), axis=1, keepdims=True)
    ce = lse - zlab

    per_row = lobj * obj + lbox * mse + lclass * ce
    o_ref[0, 0] = jnp.sum(free * per_row)


def kernel(X, yboxes, ylabels, anchors, nclasses, iou_thresh, lclass, lnoobj,
           lobj, lbox):
    B, A, H, W, C = X.shape
    hgrid = H
    wgrid = A
    NB = yboxes.shape[1]
    NJ = B * NB
    NR = A * NJ
    NC = int(nclasses)

    table = X[:, :, :, :wgrid, :].reshape(B * A * hgrid * wgrid, C)
    table = jnp.concatenate(
        [table, jnp.zeros((table.shape[0], _CP - C), jnp.float32)], axis=1)
    boxes = yboxes.reshape(NJ, 4)
    bx = boxes[:, 0]
    by = boxes[:, 1]

    G = _sc_gather(table, bx, by, A, NJ, NB, hgrid, wgrid)

    P = jnp.tile(boxes, (A, 1))
    PT = jnp.concatenate(
        [jnp.tile(bx, A)[None, :], jnp.tile(by, A)[None, :],
         jnp.zeros((6, NR), jnp.float32)], axis=0)
    A2 = jnp.repeat(anchors.astype(jnp.float32), NJ, axis=0)
    lab2 = jnp.tile(ylabels.reshape(NJ), A).reshape(NR, 1).astype(jnp.int32)
    coef = jnp.stack([jnp.float32(lobj), jnp.float32(lbox),
                      jnp.float32(lclass), jnp.float32(0.0)])

    body = functools.partial(_loss_body, NR, NJ, NC, hgrid, wgrid)
    loss = pl.pallas_call(
        body,
        out_shape=jax.ShapeDtypeStruct((1, 1), jnp.float32),
        in_specs=[
            pl.BlockSpec(memory_space=pltpu.VMEM),
            pl.BlockSpec(memory_space=pltpu.VMEM),
            pl.BlockSpec(memory_space=pltpu.VMEM),
            pl.BlockSpec(memory_space=pltpu.VMEM),
            pl.BlockSpec(memory_space=pltpu.VMEM),
            pl.BlockSpec(memory_space=pltpu.SMEM),
        ],
        out_specs=pl.BlockSpec(memory_space=pltpu.SMEM),
    )(G, P, PT, A2, lab2, coef)
    return loss.reshape(1)


# speedup vs baseline: 133.1092x; 133.1092x over previous
import functools

import jax
import jax.numpy as jnp
from jax import lax
from jax.experimental import pallas as pl
from jax.experimental.pallas import tpu as pltpu
from jax.experimental.pallas import tpu_sc as plsc

_L = 16
_CP = 128


def _sc_gather(table, bxr, byr, basev, NR, hgrid, wgrid):
    n_chunks = NR // _L
    mesh = plsc.VectorSubcoreMesh(core_axis_name="c", subcore_axis_name="s")
    n_sub = mesh.num_subcores

    @functools.partial(
        pl.kernel,
        out_type=jax.ShapeDtypeStruct((NR, _CP), jnp.float32),
        mesh=mesh,
        scratch_types=[
            pltpu.VMEM((_L,), jnp.float32),
            pltpu.VMEM((_L,), jnp.float32),
            pltpu.VMEM((_L,), jnp.int32),
            pltpu.VMEM((_L,), jnp.int32),
            pltpu.VMEM((_L, _CP), jnp.float32),
            pltpu.SemaphoreType.DMA,
        ],
    )
    def gather_k(table_hbm, bx_hbm, by_hbm, base_hbm, out_hbm,
                 bx_v, by_v, base_v, idx_v, rows_v, sem):
        wid = lax.axis_index("c") * n_sub + lax.axis_index("s")

        @pl.when(wid < n_chunks)
        def _():
            rbase = pl.multiple_of(wid * _L, _L)
            pltpu.sync_copy(bx_hbm.at[pl.ds(rbase, _L)], bx_v)
            pltpu.sync_copy(by_hbm.at[pl.ds(rbase, _L)], by_v)
            pltpu.sync_copy(base_hbm.at[pl.ds(rbase, _L)], base_v)
            yv = (by_v[...] * jnp.float32(hgrid)).astype(jnp.int32)
            xv = (bx_v[...] * jnp.float32(wgrid)).astype(jnp.int32)
            idx_v[...] = base_v[...] + yv * wgrid + xv
            pltpu.async_copy(table_hbm.at[idx_v], rows_v, sem).wait()
            pltpu.sync_copy(rows_v, out_hbm.at[pl.ds(rbase, _L)])

    return gather_k(table, bxr, byr, basev)


def _loss_body(NR, NJ, NC, hgrid, wgrid,
               g_ref, p_ref, pt_ref, a_ref, l_ref, c_ref, o_ref):
    g = g_ref[...]
    p = p_ref[...]
    pt = pt_ref[...]
    anc = a_ref[...]
    lab = l_ref[...]
    lobj, lbox, lclass = c_ref[0], c_ref[1], c_ref[2]

    bx, by = p[:, 0:1], p[:, 1:2]
    bw, bh = p[:, 2:3], p[:, 3:4]
    hf = jnp.float32(hgrid)
    wf = jnp.float32(wgrid)
    yf = jnp.floor(by * hf)
    xf = jnp.floor(bx * wf)
    cellc = yf * wf + xf
    cellr = jnp.floor(pt[1:2, :] * hf) * wf + jnp.floor(pt[0:1, :] * wf)

    jr = lax.broadcasted_iota(jnp.int32, (NR, NR), 0) % NJ
    jc = lax.broadcasted_iota(jnp.int32, (NR, NR), 1) % NJ
    clash = jnp.logical_and(cellr == cellc, jc < jr)
    free = 1.0 - jnp.max(clash.astype(jnp.float32), axis=1, keepdims=True)

    xrel = (bx - xf / wf) * wf
    yrel = (by - yf / hf) * hf
    wc = bw / anc[:, 0:1]
    hc = bh / anc[:, 1:2]

    col = lax.broadcasted_iota(jnp.int32, (NR, _CP), 1)
    z0 = jnp.sum(jnp.where(col == 0, g, 0.0), axis=1, keepdims=True)
    t = -z0
    obj = jnp.maximum(t, 0.0) + jnp.log1p(jnp.exp(-jnp.abs(t)))
    tgt = (jnp.where(col == 1, xrel, 0.0) + jnp.where(col == 2, yrel, 0.0)
           + jnp.where(col == 3, wc, 0.0) + jnp.where(col == 4, hc, 0.0))
    boxmask = jnp.logical_and(col >= 1, col <= 4)
    mse = jnp.sum(jnp.where(boxmask, (g - tgt) ** 2, 0.0),
                  axis=1, keepdims=True) * 0.25
    cmask = jnp.logical_and(col >= 5, col < 5 + NC)
    m = jnp.max(jnp.where(cmask, g, jnp.float32(-1e30)), axis=1, keepdims=True)
    se = jnp.sum(jnp.where(cmask, jnp.exp(g - m), 0.0), axis=1, keepdims=True)
    lse = m + jnp.log(se)
    zlab = jnp.sum(jnp.where(col == lab + 5, g, 0.0), axis=1, keepdims=True)
    ce = lse - zlab

    per_row = lobj * obj + lbox * mse + lclass * ce
    o_ref[0, 0] = jnp.sum(free * per_row)


def kernel(X, yboxes, ylabels, anchors, nclasses, iou_thresh, lclass, lnoobj,
           lobj, lbox):
    B, A, H, W, C = X.shape
    hgrid = H
    wgrid = A
    NB = yboxes.shape[1]
    NJ = B * NB
    NR = A * NJ
    NC = C - 5

    table = X[:, :, :, :wgrid, :].reshape(B * A * hgrid * wgrid, C)
    table = jnp.concatenate(
        [table, jnp.zeros((table.shape[0], _CP - C), jnp.float32)], axis=1)
    boxes = yboxes.reshape(NJ, 4)
    bx = boxes[:, 0]
    by = boxes[:, 1]

    ja = jnp.arange(NR, dtype=jnp.int32)
    av = ja // NJ
    bv = (ja % NJ) // NB
    basev = (bv * A + av) * (hgrid * wgrid)
    bxr = jnp.tile(bx, A)
    byr = jnp.tile(by, A)

    G = _sc_gather(table, bxr, byr, basev, NR, hgrid, wgrid)

    P = jnp.tile(boxes, (A, 1))
    PT = jnp.concatenate(
        [jnp.tile(bx, A)[None, :], jnp.tile(by, A)[None, :],
         jnp.zeros((6, NR), jnp.float32)], axis=0)
    A2 = jnp.repeat(anchors.astype(jnp.float32), NJ, axis=0)
    lab2 = jnp.tile(ylabels.reshape(NJ), A).reshape(NR, 1).astype(jnp.int32)
    coef = jnp.stack([jnp.float32(lobj), jnp.float32(lbox),
                      jnp.float32(lclass), jnp.float32(0.0)])

    body = functools.partial(_loss_body, NR, NJ, NC, hgrid, wgrid)
    loss = pl.pallas_call(
        body,
        out_shape=jax.ShapeDtypeStruct((1, 1), jnp.float32),
        in_specs=[
            pl.BlockSpec(memory_space=pltpu.VMEM),
            pl.BlockSpec(memory_space=pltpu.VMEM),
            pl.BlockSpec(memory_space=pltpu.VMEM),
            pl.BlockSpec(memory_space=pltpu.VMEM),
            pl.BlockSpec(memory_space=pltpu.VMEM),
            pl.BlockSpec(memory_space=pltpu.SMEM),
        ],
        out_specs=pl.BlockSpec(memory_space=pltpu.SMEM),
    )(G, P, PT, A2, lab2, coef)
    return loss.reshape(1)


# speedup vs baseline: 145.3168x; 1.0917x over previous
import functools

import jax
import jax.numpy as jnp
from jax import lax
from jax.experimental import pallas as pl
from jax.experimental.pallas import tpu as pltpu
from jax.experimental.pallas import tpu_sc as plsc

_L = 16
_CP = 128
_CELL = 256


def _sc_gather(table, comb, NR, hgrid, wgrid):
    n_chunks = NR // _L
    mesh = plsc.VectorSubcoreMesh(core_axis_name="c", subcore_axis_name="s")
    n_sub = mesh.num_subcores

    @functools.partial(
        pl.kernel,
        out_type=jax.ShapeDtypeStruct((NR, _CP), jnp.float32),
        mesh=mesh,
        scratch_types=[
            pltpu.VMEM((8, _L), jnp.float32),
            pltpu.VMEM((_L,), jnp.int32),
            pltpu.VMEM((_L, _CP), jnp.float32),
            pltpu.SemaphoreType.DMA,
        ],
    )
    def gather_k(table_hbm, comb_hbm, out_hbm, comb_v, idx_v, rows_v, sem):
        wid = lax.axis_index("c") * n_sub + lax.axis_index("s")

        @pl.when(wid < n_chunks)
        def _():
            cbase = pl.multiple_of(wid * 8, 8)
            pltpu.sync_copy(comb_hbm.at[pl.ds(cbase, 8)], comb_v)
            bxv = comb_v[0]
            byv = comb_v[1]
            basev = comb_v[2].astype(jnp.int32)
            yv = (byv * jnp.float32(hgrid)).astype(jnp.int32)
            xv = (bxv * jnp.float32(wgrid)).astype(jnp.int32)
            idx_v[...] = basev + yv * wgrid + xv
            pltpu.async_copy(table_hbm.at[idx_v], rows_v, sem).wait()
            rbase = pl.multiple_of(wid * _L, _L)
            pltpu.sync_copy(rows_v, out_hbm.at[pl.ds(rbase, _L)])

    return gather_k(table, comb)


def _loss_body(NR, NJ, NC, hgrid, wgrid, g_ref, pk_ref, c_ref, o_ref):
    g = g_ref[...]
    pk = pk_ref[...]
    lobj, lbox, lclass = c_ref[0], c_ref[1], c_ref[2]

    bx, by = pk[:, 0:1], pk[:, 1:2]
    bw, bh = pk[:, 2:3], pk[:, 3:4]
    aw, ah = pk[:, 4:5], pk[:, 5:6]
    labf = pk[:, 6:7]
    hf = jnp.float32(hgrid)
    wf = jnp.float32(wgrid)
    yf = jnp.floor(by * hf)
    xf = jnp.floor(bx * wf)
    cellc = yf * wf + xf

    cellcol = lax.broadcasted_iota(jnp.int32, (NR, _CELL), 1)
    oh = (cellcol == cellc.astype(jnp.int32)).astype(jnp.float32)
    eq = lax.dot_general(oh, oh, (((1,), (1,)), ((), ())),
                         preferred_element_type=jnp.float32)
    jr = lax.broadcasted_iota(jnp.int32, (NR, NR), 0) % NJ
    jc = lax.broadcasted_iota(jnp.int32, (NR, NR), 1) % NJ
    clash = jnp.logical_and(eq > 0.0, jc < jr)
    free = 1.0 - jnp.max(clash.astype(jnp.float32), axis=1, keepdims=True)

    xrel = (bx - xf / wf) * wf
    yrel = (by - yf / hf) * hf
    wc = bw / aw
    hc = bh / ah

    col = lax.broadcasted_iota(jnp.int32, (NR, _CP), 1)
    z0 = jnp.sum(jnp.where(col == 0, g, 0.0), axis=1, keepdims=True)
    t = -z0
    obj = jnp.maximum(t, 0.0) + jnp.log1p(jnp.exp(-jnp.abs(t)))
    tgt = (jnp.where(col == 1, xrel, 0.0) + jnp.where(col == 2, yrel, 0.0)
           + jnp.where(col == 3, wc, 0.0) + jnp.where(col == 4, hc, 0.0))
    boxmask = jnp.logical_and(col >= 1, col <= 4)
    mse = jnp.sum(jnp.where(boxmask, (g - tgt) ** 2, 0.0),
                  axis=1, keepdims=True) * 0.25
    cmask = jnp.logical_and(col >= 5, col < 5 + NC)
    m = jnp.max(jnp.where(cmask, g, jnp.float32(-1e30)), axis=1, keepdims=True)
    se = jnp.sum(jnp.where(cmask, jnp.exp(g - m), 0.0), axis=1, keepdims=True)
    lse = m + jnp.log(se)
    zlab = jnp.sum(jnp.where(col == labf.astype(jnp.int32) + 5, g, 0.0),
                   axis=1, keepdims=True)
    ce = lse - zlab

    per_row = lobj * obj + lbox * mse + lclass * ce
    o_ref[0, 0] = jnp.sum(free * per_row)


def kernel(X, yboxes, ylabels, anchors, nclasses, iou_thresh, lclass, lnoobj,
           lobj, lbox):
    B, A, H, W, C = X.shape
    hgrid = H
    wgrid = A
    NB = yboxes.shape[1]
    NJ = B * NB
    NR = A * NJ
    NC = C - 5
    n_chunks = NR // _L

    table = X[:, :, :, :wgrid, :].reshape(B * A * hgrid * wgrid, C)
    table = jnp.concatenate(
        [table, jnp.zeros((table.shape[0], _CP - C), jnp.float32)], axis=1)
    boxes = yboxes.reshape(NJ, 4)
    bx = boxes[:, 0]
    by = boxes[:, 1]

    ja = jnp.arange(NR, dtype=jnp.int32)
    av = ja // NJ
    bv = (ja % NJ) // NB
    basev = (bv * A + av) * (hgrid * wgrid)
    basef = basev.astype(jnp.float32)
    comb = jnp.concatenate(
        [jnp.tile(bx, A).reshape(n_chunks, 1, _L),
         jnp.tile(by, A).reshape(n_chunks, 1, _L),
         basef.reshape(n_chunks, 1, _L),
         jnp.zeros((n_chunks, 5, _L), jnp.float32)], axis=1
    ).reshape(n_chunks * 8, _L)

    G = _sc_gather(table, comb, NR, hgrid, wgrid)

    labf = ylabels.reshape(NJ).astype(jnp.float32)
    PK = jnp.concatenate(
        [jnp.tile(boxes, (A, 1)),
         jnp.repeat(anchors.astype(jnp.float32), NJ, axis=0),
         jnp.tile(labf, A)[:, None],
         jnp.zeros((NR, 1), jnp.float32)], axis=1)
    coef = jnp.stack([jnp.float32(lobj), jnp.float32(lbox),
                      jnp.float32(lclass), jnp.float32(0.0)])

    body = functools.partial(_loss_body, NR, NJ, NC, hgrid, wgrid)
    loss = pl.pallas_call(
        body,
        out_shape=jax.ShapeDtypeStruct((1, 1), jnp.float32),
        in_specs=[
            pl.BlockSpec(memory_space=pltpu.VMEM),
            pl.BlockSpec(memory_space=pltpu.VMEM),
            pl.BlockSpec(memory_space=pltpu.SMEM),
        ],
        out_specs=pl.BlockSpec(memory_space=pltpu.SMEM),
    )(G, PK, coef)
    return loss.reshape(1)


# speedup vs baseline: 153.3515x; 1.0553x over previous
import functools

import jax
import jax.numpy as jnp
from jax import lax
from jax.experimental import pallas as pl
from jax.experimental.pallas import tpu as pltpu
from jax.experimental.pallas import tpu_sc as plsc

_L = 16
_CP = 128
_CELL = 256


def _sc_gather(table, comb, NR, hgrid, wgrid):
    n_chunks = NR // _L
    mesh = plsc.VectorSubcoreMesh(core_axis_name="c", subcore_axis_name="s",
                                  num_cores=1)
    n_sub = mesh.num_subcores

    @functools.partial(
        pl.kernel,
        out_type=jax.ShapeDtypeStruct((NR, _CP), jnp.float32),
        mesh=mesh,
        scratch_types=[
            pltpu.VMEM((8, _L), jnp.float32),
            pltpu.VMEM((_L,), jnp.int32),
            pltpu.VMEM((_L, _CP), jnp.float32),
            pltpu.SemaphoreType.DMA,
        ],
    )
    def gather_k(table_hbm, comb_hbm, out_hbm, comb_v, idx_v, rows_v, sem):
        wid = lax.axis_index("c") * n_sub + lax.axis_index("s")

        @pl.when(wid < n_chunks)
        def _():
            cbase = pl.multiple_of(wid * 8, 8)
            pltpu.sync_copy(comb_hbm.at[pl.ds(cbase, 8)], comb_v)
            bxv = comb_v[0]
            byv = comb_v[1]
            basev = comb_v[2].astype(jnp.int32)
            yv = (byv * jnp.float32(hgrid)).astype(jnp.int32)
            xv = (bxv * jnp.float32(wgrid)).astype(jnp.int32)
            idx_v[...] = basev + yv * wgrid + xv
            pltpu.async_copy(table_hbm.at[idx_v], rows_v, sem).wait()
            rbase = pl.multiple_of(wid * _L, _L)
            pltpu.sync_copy(rows_v, out_hbm.at[pl.ds(rbase, _L)])

    return gather_k(table, comb)


def _loss_body(NR, NJ, NC, hgrid, wgrid, g_ref, pk_ref, c_ref, o_ref):
    g = g_ref[...]
    pk = pk_ref[...]
    lobj, lbox, lclass = c_ref[0], c_ref[1], c_ref[2]

    bx, by = pk[:, 0:1], pk[:, 1:2]
    bw, bh = pk[:, 2:3], pk[:, 3:4]
    aw, ah = pk[:, 4:5], pk[:, 5:6]
    labf = pk[:, 6:7]
    hf = jnp.float32(hgrid)
    wf = jnp.float32(wgrid)
    yf = jnp.floor(by * hf)
    xf = jnp.floor(bx * wf)
    cellc = yf * wf + xf

    cellcol = lax.broadcasted_iota(jnp.int32, (NR, _CELL), 1)
    oh = (cellcol == cellc.astype(jnp.int32)).astype(jnp.float32)
    eq = lax.dot_general(oh, oh, (((1,), (1,)), ((), ())),
                         preferred_element_type=jnp.float32)
    jr = lax.broadcasted_iota(jnp.int32, (NR, NR), 0) % NJ
    jc = lax.broadcasted_iota(jnp.int32, (NR, NR), 1) % NJ
    clash = jnp.logical_and(eq > 0.0, jc < jr)
    free = 1.0 - jnp.max(clash.astype(jnp.float32), axis=1, keepdims=True)

    xrel = (bx - xf / wf) * wf
    yrel = (by - yf / hf) * hf
    wc = bw / aw
    hc = bh / ah

    col = lax.broadcasted_iota(jnp.int32, (NR, _CP), 1)
    z0 = jnp.sum(jnp.where(col == 0, g, 0.0), axis=1, keepdims=True)
    t = -z0
    obj = jnp.maximum(t, 0.0) + jnp.log1p(jnp.exp(-jnp.abs(t)))
    tgt = (jnp.where(col == 1, xrel, 0.0) + jnp.where(col == 2, yrel, 0.0)
           + jnp.where(col == 3, wc, 0.0) + jnp.where(col == 4, hc, 0.0))
    boxmask = jnp.logical_and(col >= 1, col <= 4)
    mse = jnp.sum(jnp.where(boxmask, (g - tgt) ** 2, 0.0),
                  axis=1, keepdims=True) * 0.25
    cmask = jnp.logical_and(col >= 5, col < 5 + NC)
    m = jnp.max(jnp.where(cmask, g, jnp.float32(-1e30)), axis=1, keepdims=True)
    se = jnp.sum(jnp.where(cmask, jnp.exp(g - m), 0.0), axis=1, keepdims=True)
    lse = m + jnp.log(se)
    zlab = jnp.sum(jnp.where(col == labf.astype(jnp.int32) + 5, g, 0.0),
                   axis=1, keepdims=True)
    ce = lse - zlab

    per_row = lobj * obj + lbox * mse + lclass * ce
    o_ref[0, 0] = jnp.sum(free * per_row)


def kernel(X, yboxes, ylabels, anchors, nclasses, iou_thresh, lclass, lnoobj,
           lobj, lbox):
    B, A, H, W, C = X.shape
    hgrid = H
    wgrid = A
    NB = yboxes.shape[1]
    NJ = B * NB
    NR = A * NJ
    NC = C - 5
    n_chunks = NR // _L

    table = X[:, :, :, :wgrid, :].reshape(B * A * hgrid * wgrid, C)
    table = jnp.concatenate(
        [table, jnp.zeros((table.shape[0], _CP - C), jnp.float32)], axis=1)
    boxes = yboxes.reshape(NJ, 4)
    bx = boxes[:, 0]
    by = boxes[:, 1]

    ja = jnp.arange(NR, dtype=jnp.int32)
    av = ja // NJ
    bv = (ja % NJ) // NB
    basev = (bv * A + av) * (hgrid * wgrid)
    basef = basev.astype(jnp.float32)
    comb = jnp.concatenate(
        [jnp.tile(bx, A).reshape(n_chunks, 1, _L),
         jnp.tile(by, A).reshape(n_chunks, 1, _L),
         basef.reshape(n_chunks, 1, _L),
         jnp.zeros((n_chunks, 5, _L), jnp.float32)], axis=1
    ).reshape(n_chunks * 8, _L)

    G = _sc_gather(table, comb, NR, hgrid, wgrid)

    labf = ylabels.reshape(NJ).astype(jnp.float32)
    PK = jnp.concatenate(
        [jnp.tile(boxes, (A, 1)),
         jnp.repeat(anchors.astype(jnp.float32), NJ, axis=0),
         jnp.tile(labf, A)[:, None],
         jnp.zeros((NR, 1), jnp.float32)], axis=1)
    coef = jnp.stack([jnp.float32(lobj), jnp.float32(lbox),
                      jnp.float32(lclass), jnp.float32(0.0)])

    body = functools.partial(_loss_body, NR, NJ, NC, hgrid, wgrid)
    loss = pl.pallas_call(
        body,
        out_shape=jax.ShapeDtypeStruct((1, 1), jnp.float32),
        in_specs=[
            pl.BlockSpec(memory_space=pltpu.VMEM),
            pl.BlockSpec(memory_space=pltpu.VMEM),
            pl.BlockSpec(memory_space=pltpu.SMEM),
        ],
        out_specs=pl.BlockSpec(memory_space=pltpu.SMEM),
    )(G, PK, coef)
    return loss.reshape(1)
